# baseline (speedup 1.0000x reference)
import functools

import jax
import jax.numpy as jnp
from jax import lax
from jax.experimental import pallas as pl
from jax.experimental.pallas import tpu as pltpu
from jax.experimental.pallas import tpu_sc as plsc

N = 10000
DF = 256
HID = 256
HEADS = 3
OUT = 256
E_RAW = 160000
E_TOT = E_RAW + N

NC, NS = 2, 16
NW = NC * NS

PER = 5376
EP = PER * NW
BK = 64
NB = PER // BK
NI = PER // 16

PERF = 5120
EPF = PERF * NW
NBF = PERF // BK

ROWS_PT = 632
ROWS_TAIL = N - (NS - 1) * ROWS_PT


def _pad128(x):
    return (x + 127) // 128 * 128


def _mesh():
    return plsc.VectorSubcoreMesh(core_axis_name="c", subcore_axis_name="s")


def _mm_logits(x, W, As, Ad, heads, ch, rb=1000):
    m, k = x.shape
    M = W.shape[1]

    def body(x_ref, w_ref, as_ref, ad_ref, *outs):
        h = jnp.dot(x_ref[...], w_ref[...], preferred_element_type=jnp.float32)
        for j in range(ch):
            outs[j][...] = h[:, j * 128:(j + 1) * 128]
        outs[ch][...] = jnp.dot(h, as_ref[...], preferred_element_type=jnp.float32)
        outs[ch + 1][...] = jnp.dot(h, ad_ref[...], preferred_element_type=jnp.float32)

    res = pl.pallas_call(
        body,
        grid=(m // rb,),
        in_specs=[pl.BlockSpec((rb, k), lambda i: (i, 0)),
                  pl.BlockSpec((k, M), lambda i: (0, 0)),
                  pl.BlockSpec((M, heads), lambda i: (0, 0)),
                  pl.BlockSpec((M, heads), lambda i: (0, 0))],
        out_specs=[pl.BlockSpec((rb, 128), lambda i: (i, 0))] * ch
        + [pl.BlockSpec((rb, heads), lambda i: (i, 0))] * 2,
        out_shape=[jax.ShapeDtypeStruct((m, 128), jnp.float32)] * ch
        + [jax.ShapeDtypeStruct((m, heads), jnp.float32)] * 2,
    )(x, W, As, Ad)
    return list(res[:ch]), res[ch], res[ch + 1]


def _mm2_logits(parts, b1c, W2, As, Ad, rb=1000):
    def body(*refs):
        prefs = refs[:6]
        b_ref, w_ref, as_ref, ad_ref = refs[6:10]
        outs = refs[10:]
        acc = jnp.zeros((rb, OUT), jnp.float32)
        for p in range(6):
            xp = prefs[p][0] + prefs[p][1] + b_ref[p][None, :]
            xp = jnp.maximum(xp, 0.0)
            acc = acc + jnp.dot(xp, w_ref[p * 128:(p + 1) * 128, :],
                                preferred_element_type=jnp.float32)
        for j in range(2):
            outs[j][...] = acc[:, j * 128:(j + 1) * 128]
        outs[2][...] = jnp.dot(acc, as_ref[...], preferred_element_type=jnp.float32)
        outs[3][...] = jnp.dot(acc, ad_ref[...], preferred_element_type=jnp.float32)

    res = pl.pallas_call(
        body,
        grid=(N // rb,),
        in_specs=[pl.BlockSpec((NC, rb, 128), lambda i: (0, i, 0))] * 6
        + [pl.BlockSpec((6, 128), lambda i: (0, 0)),
           pl.BlockSpec((HEADS * HID, OUT), lambda i: (0, 0)),
           pl.BlockSpec((OUT, 1), lambda i: (0, 0)),
           pl.BlockSpec((OUT, 1), lambda i: (0, 0))],
        out_specs=[pl.BlockSpec((rb, 128), lambda i: (i, 0))] * 2
        + [pl.BlockSpec((rb, 1), lambda i: (i, 0))] * 2,
        out_shape=[jax.ShapeDtypeStruct((N, 128), jnp.float32)] * 2
        + [jax.ShapeDtypeStruct((N, 1), jnp.float32)] * 2,
    )(*parts, b1c, W2, As, Ad)
    return list(res[:2]), res[2], res[3]


def _den_recip(parts):
    nh = parts.shape[1]

    def body(p_ref, o_ref):
        s = jnp.sum(p_ref[...], axis=0)
        o_ref[...] = 1.0 / (s + 1e-16)

    return pl.pallas_call(
        body,
        in_specs=[pl.BlockSpec((NW, nh), lambda: (0, 0))],
        out_specs=pl.BlockSpec((nh,), lambda: (0,)),
        out_shape=jax.ShapeDtypeStruct((nh,), jnp.float32),
    )(parts)


def _assemble_h2(parts, b2c, rb=1000):
    def body(p0, p1, b_ref, o_ref):
        o_ref[:, 0:128] = p0[0] + p0[1] + b_ref[0][None, :]
        o_ref[:, 128:256] = p1[0] + p1[1] + b_ref[1][None, :]

    return pl.pallas_call(
        body,
        grid=(N // rb,),
        in_specs=[pl.BlockSpec((NC, rb, 128), lambda i: (0, i, 0))] * 2
        + [pl.BlockSpec((2, 128), lambda i: (0, 0))],
        out_specs=pl.BlockSpec((rb, OUT), lambda i: (i, 0)),
        out_shape=jax.ShapeDtypeStruct((N, OUT), jnp.float32),
    )(parts[0], parts[1], b2c)


def _sc_edge_softmax_num(es_f, ed_f, srcp, dstp, zeros_nh, heads):
    nh = N * heads
    nhp = _pad128(nh)

    @functools.partial(
        pl.kernel, mesh=_mesh(),
        compiler_params=pltpu.CompilerParams(needs_layout_passes=False),
        out_type=[jax.ShapeDtypeStruct((heads * EP,), jnp.float32),
                  jax.ShapeDtypeStruct((NW * nhp,), jnp.float32)],
        scratch_types=[pltpu.VMEM((nh,), jnp.float32),
                       pltpu.VMEM((nh,), jnp.float32),
                       pltpu.VMEM((nhp,), jnp.float32),
                       pltpu.VMEM((PER,), jnp.int32),
                       pltpu.VMEM((PER,), jnp.int32),
                       pltpu.VMEM((heads * PER,), jnp.float32)],
    )
    def k(es_hbm, ed_hbm, src_hbm, dst_hbm, z_hbm, ex_hbm, dp_hbm,
          es_t, ed_t, den_t, src_v, dst_v, ex_v):
        wid = lax.axis_index("s") * NC + lax.axis_index("c")
        base = wid * PER
        pltpu.sync_copy(es_hbm, es_t)
        pltpu.sync_copy(ed_hbm, ed_t)
        pltpu.sync_copy(z_hbm, den_t)
        pltpu.sync_copy(src_hbm.at[pl.ds(base, PER)], src_v)
        pltpu.sync_copy(dst_hbm.at[pl.ds(base, PER)], dst_v)

        def body(i, carry):
            s = src_v[pl.ds(i * 16, 16)]
            d = dst_v[pl.ds(i * 16, 16)]
            gid = base + i * 16 + lax.iota(jnp.int32, 16)
            valid = gid < E_TOT
            for h in range(heads):
                es_g = plsc.load_gather(es_t, [s * heads + h])
                ed_g = plsc.load_gather(ed_t, [d * heads + h])
                e = es_g + ed_g
                e = jnp.maximum(e, 0.2 * e)
                ex = jnp.where(valid, jnp.exp(e), 0.0)
                ex_v[pl.ds(h * PER + i * 16, 16)] = ex
                plsc.addupdate_scatter(den_t, [d * heads + h], ex)
            return carry

        lax.fori_loop(0, NI, body, 0)
        for h in range(heads):
            pltpu.sync_copy(ex_v.at[pl.ds(h * PER, PER)],
                            ex_hbm.at[pl.ds(h * EP + base, PER)])
        pltpu.sync_copy(den_t, dp_hbm.at[pl.ds(wid * nhp, nhp)])

    return k(es_f, ed_f, srcp, dstp, zeros_nh)


def _sc_alpha(ex_f, recip_f, dstp, heads):
    nhp = recip_f.shape[0]

    @functools.partial(
        pl.kernel, mesh=_mesh(),
        compiler_params=pltpu.CompilerParams(needs_layout_passes=False),
        out_type=jax.ShapeDtypeStruct((heads * EP,), jnp.float32),
        scratch_types=[pltpu.VMEM((nhp,), jnp.float32),
                       pltpu.VMEM((PER,), jnp.int32),
                       pltpu.VMEM((heads * PER,), jnp.float32)],
    )
    def k(ex_hbm, recip_hbm, dst_hbm, al_hbm, recip_t, dst_v, ex_v):
        wid = lax.axis_index("s") * NC + lax.axis_index("c")
        base = wid * PER
        pltpu.sync_copy(recip_hbm, recip_t)
        pltpu.sync_copy(dst_hbm.at[pl.ds(base, PER)], dst_v)
        for h in range(heads):
            pltpu.sync_copy(ex_hbm.at[pl.ds(h * EP + base, PER)],
                            ex_v.at[pl.ds(h * PER, PER)])

        def abody(i, carry):
            d = dst_v[pl.ds(i * 16, 16)]
            for h in range(heads):
                rd = plsc.load_gather(recip_t, [d * heads + h])
                ex_v[pl.ds(h * PER + i * 16, 16)] = \
                    ex_v[pl.ds(h * PER + i * 16, 16)] * rd
            return carry

        lax.fori_loop(0, NI, abody, 0)
        for h in range(heads):
            pltpu.sync_copy(ex_v.at[pl.ds(h * PER, PER)],
                            al_hbm.at[pl.ds(h * EP + base, PER)])

    return k(ex_f, recip_f, dstp)


def _sc_aggregate(h_chunks, src2, dst2, alpha_f, zslab, heads, ch):
    @functools.partial(
        pl.kernel, mesh=_mesh(),
        compiler_params=pltpu.CompilerParams(needs_layout_passes=False),
        out_type=[jax.ShapeDtypeStruct((NC, N, 128), jnp.float32)
                  for _ in range(ch)],
        scratch_types=[pltpu.VMEM_SHARED((N, 128), jnp.float32),
                       pltpu.VMEM((NB, BK), jnp.int32),
                       pltpu.VMEM((NB, BK), jnp.int32),
                       pltpu.VMEM((PER // 2,), jnp.float32),
                       pltpu.VMEM((BK, 128), jnp.float32),
                       pltpu.VMEM((BK, 128), jnp.float32),
                       pltpu.VMEM((BK, 128), jnp.float32),
                       pltpu.SemaphoreType.DMA,
                       pltpu.SemaphoreType.DMA,
                       pltpu.SemaphoreType.DMA,
                       pltpu.SemaphoreType.DMA,
                       pltpu.SemaphoreType.DMA,
                       pltpu.SemaphoreType.DMA],
    )
    def k(*refs):
        hs = refs[:ch]
        src2_h, dst2_h, al_h, z_h = refs[ch:ch + 4]
        outs = refs[ch + 4:2 * ch + 4]
        (slab, src2_v, dst2_v, alpha_v, rows0, rows1, rows2,
         g0, g1, g2, s0, s1, s2) = refs[2 * ch + 4:]
        bufs = (rows0, rows1, rows2)
        gsems = (g0, g1, g2)
        ssems = (s0, s1, s2)
        cid = lax.axis_index("c")
        sid = lax.axis_index("s")
        wid = sid * NC + cid
        base = wid * PER
        pltpu.sync_copy(src2_h.at[wid], src2_v)
        pltpu.sync_copy(dst2_h.at[wid], dst2_v)

        r0 = sid * ROWS_PT
        HALF = PER // 2
        for c in range(ch):
            h0 = c // 2
            pltpu.sync_copy(al_h.at[pl.ds(h0 * EP + base, HALF)], alpha_v)

            def gather(b, kbuf):
                pltpu.async_copy(hs[c].at[src2_v.at[b]], bufs[kbuf],
                                 gsems[kbuf])

            def wait_gather(kbuf):
                pltpu.make_async_copy(z_h.at[pl.ds(0, BK), :], bufs[kbuf],
                                      gsems[kbuf]).wait()

            def scatter(b, kbuf):
                pltpu.async_copy(bufs[kbuf], slab.at[dst2_v.at[b]],
                                 ssems[kbuf], add=True)

            def wait_scatter(kbuf):
                pltpu.make_async_copy(bufs[kbuf], slab.at[pl.ds(0, BK), :],
                                      ssems[kbuf]).wait()

            def scale(b, kbuf):
                buf = bufs[kbuf]

                rel = lax.rem(b * BK, HALF)

                @plsc.parallel_loop(0, BK // 16, 1, unroll=2)
                def _(r16):
                    a_vec = alpha_v[pl.ds(rel + r16 * 16, 16)]
                    for rr in range(16):
                        row = r16 * 16 + rr
                        av = jnp.full((16,), a_vec[rr], jnp.float32)
                        for j in range(8):
                            buf[row, pl.ds(j * 16, 16)] = \
                                buf[row, pl.ds(j * 16, 16)] * av

            @pl.when(sid < NS - 1)
            def _():
                pltpu.sync_copy(z_h.at[pl.ds(r0, ROWS_PT), :],
                                slab.at[pl.ds(r0, ROWS_PT), :])

            @pl.when(sid == NS - 1)
            def _():
                pltpu.sync_copy(z_h.at[pl.ds(r0, ROWS_TAIL), :],
                                slab.at[pl.ds(r0, ROWS_TAIL), :])

            plsc.subcore_barrier()

            gather(0, 0)

            def pbody(t, carry):
                @pl.when(t == NB // 3 // 2)
                def _():
                    pltpu.sync_copy(
                        al_h.at[pl.ds(h0 * EP + base + HALF, HALF)], alpha_v)

                for ph in range(3):
                    b = 3 * t + ph
                    k1 = (ph + 1) % 3
                    wait_gather(ph)
                    if ph == 2:
                        wait_scatter(k1)

                        @pl.when(t < NB // 3 - 1)
                        def _():
                            gather(b + 1, k1)
                    else:

                        @pl.when(t > 0)
                        def _():
                            wait_scatter(k1)

                        gather(b + 1, k1)
                    scale(b, ph)
                    scatter(b, ph)
                return carry

            lax.fori_loop(0, NB // 3, pbody, 0)
            wait_scatter(1)
            wait_scatter(2)
            plsc.subcore_barrier()

            @pl.when(sid < NS - 1)
            def _():
                pltpu.sync_copy(slab.at[pl.ds(r0, ROWS_PT), :],
                                outs[c].at[cid, pl.ds(r0, ROWS_PT), :])

            @pl.when(sid == NS - 1)
            def _():
                pltpu.sync_copy(slab.at[pl.ds(r0, ROWS_TAIL), :],
                                outs[c].at[cid, pl.ds(r0, ROWS_TAIL), :])

            plsc.subcore_barrier()

    return k(*h_chunks, src2, dst2, alpha_f, zslab)


def _sc_final(h2, e0r, e1r):
    @functools.partial(
        pl.kernel, mesh=_mesh(),
        compiler_params=pltpu.CompilerParams(needs_layout_passes=False),
        out_type=jax.ShapeDtypeStruct((E_RAW,), jnp.float32),
        scratch_types=[pltpu.VMEM((NBF, BK), jnp.int32),
                       pltpu.VMEM((NBF, BK), jnp.int32),
                       pltpu.VMEM((BK, OUT), jnp.float32),
                       pltpu.VMEM((BK, OUT), jnp.float32),
                       pltpu.VMEM((BK, OUT), jnp.float32),
                       pltpu.VMEM((BK, OUT), jnp.float32),
                       pltpu.VMEM((PERF,), jnp.float32),
                       pltpu.SemaphoreType.DMA,
                       pltpu.SemaphoreType.DMA],
    )
    def k(h2_h, e0_h, e1_h, o_h, e0_v, e1_v, ra0, rb0, ra1, rb1,
          res_v, sem0, sem1):
        wid = lax.axis_index("s") * NC + lax.axis_index("c")
        abufs = (ra0, ra1)
        bbufs = (rb0, rb1)
        sems = (sem0, sem1)
        pltpu.sync_copy(e0_h.at[wid], e0_v)
        pltpu.sync_copy(e1_h.at[wid], e1_v)
        lane = lax.iota(jnp.int32, 16)

        def gather(b, kbuf):
            pltpu.async_copy(h2_h.at[e0_v.at[b]], abufs[kbuf], sems[kbuf])
            pltpu.async_copy(h2_h.at[e1_v.at[b]], bbufs[kbuf], sems[kbuf])

        def wait_pair(kbuf):
            pltpu.make_async_copy(h2_h.at[pl.ds(0, BK)], abufs[kbuf],
                                  sems[kbuf]).wait()
            pltpu.make_async_copy(h2_h.at[pl.ds(0, BK)], bbufs[kbuf],
                                  sems[kbuf]).wait()

        def dot(b, kbuf):
            av = abufs[kbuf]
            bv = bbufs[kbuf]

            @plsc.parallel_loop(0, BK // 16, 1, unroll=2)
            def _(r16):
                resvec = jnp.zeros((16,), jnp.float32)
                for rr in range(16):
                    row = r16 * 16 + rr
                    acc = jnp.zeros((16,), jnp.float32)
                    for j in range(OUT // 16):
                        acc = acc + (av[row, pl.ds(j * 16, 16)] *
                                     bv[row, pl.ds(j * 16, 16)])
                    resvec = resvec + jnp.where(lane == rr, jnp.sum(acc), 0.0)
                res_v[pl.ds(b * BK + r16 * 16, 16)] = resvec

        gather(0, 0)

        def pbody(i, carry):
            b0 = 2 * i
            b1 = 2 * i + 1
            wait_pair(0)
            gather(b1, 1)
            dot(b0, 0)
            wait_pair(1)

            @pl.when(i < NBF // 2 - 1)
            def _():
                gather(b0 + 2, 0)

            dot(b1, 1)
            return carry

        lax.fori_loop(0, NBF // 2, pbody, 0)

        @plsc.parallel_loop(0, PERF // 16, 1, unroll=4)
        def _(i):
            x = res_v[pl.ds(i * 16, 16)]
            res_v[pl.ds(i * 16, 16)] = 1.0 / (1.0 + jnp.exp(-x))

        @pl.when(wid < NW - 1)
        def _():
            pltpu.sync_copy(res_v, o_h.at[pl.ds(wid * PERF, PERF)])

        @pl.when(wid == NW - 1)
        def _():
            tail = E_RAW - (NW - 1) * PERF
            pltpu.sync_copy(res_v.at[pl.ds(0, tail)],
                            o_h.at[pl.ds((NW - 1) * PERF, tail)])

    return k(h2, e0r, e1r)


def kernel(Features, A, E, W1, a1_src, a1_dst, b1, W2, a2_src, a2_dst, b2):
    i32 = jnp.int32
    loops = jnp.arange(N, dtype=i32)
    zpad = jnp.zeros((EP - E_TOT,), i32)
    srcp = jnp.concatenate([A[0].astype(i32), loops, zpad])
    dstp = jnp.concatenate([A[1].astype(i32), loops, zpad])
    src2 = srcp.reshape(NW, NB, BK)
    dst2 = dstp.reshape(NW, NB, BK)

    eye = jnp.eye(HEADS, dtype=jnp.float32)
    As1 = (a1_src[:, :, None] * eye[:, None, :]).reshape(HEADS * HID, HEADS)
    Ad1 = (a1_dst[:, :, None] * eye[:, None, :]).reshape(HEADS * HID, HEADS)
    As2 = a2_src.reshape(OUT, 1)
    Ad2 = a2_dst.reshape(OUT, 1)

    zeros_nh1 = jnp.zeros((_pad128(N * HEADS),), jnp.float32)
    zeros_nh2 = jnp.zeros((_pad128(N),), jnp.float32)
    zslab = jnp.zeros((N, 128), jnp.float32)

    hc1, es1, ed1 = _mm_logits(Features, W1, As1, Ad1, HEADS, 6)
    ex1, dp1 = _sc_edge_softmax_num(es1.reshape(-1), ed1.reshape(-1),
                                    srcp, dstp, zeros_nh1, HEADS)
    recip1 = _den_recip(dp1.reshape(NW, _pad128(N * HEADS)))
    alpha1 = _sc_alpha(ex1, recip1, dstp, HEADS)
    parts1 = _sc_aggregate(hc1, src2, dst2, alpha1, zslab, HEADS, 6)

    hc2, es2, ed2 = _mm2_logits(parts1, b1.reshape(6, 128), W2, As2, Ad2)
    ex2, dp2 = _sc_edge_softmax_num(es2.reshape(-1), ed2.reshape(-1),
                                    srcp, dstp, zeros_nh2, 1)
    recip2 = _den_recip(dp2.reshape(NW, _pad128(N)))
    alpha2 = _sc_alpha(ex2, recip2, dstp, 1)
    parts2 = _sc_aggregate(hc2, src2, dst2, alpha2, zslab, 1, 2)
    h2 = _assemble_h2(parts2, b2.reshape(2, 128))

    fpad = jnp.zeros((EPF - E_RAW,), i32)
    e0r = jnp.concatenate([E[0].astype(i32), fpad]).reshape(NW, NBF, BK)
    e1r = jnp.concatenate([E[1].astype(i32), fpad]).reshape(NW, NBF, BK)
    return _sc_final(h2, e0r, e1r)


# speedup vs baseline: 1.0190x; 1.0190x over previous
import functools

import jax
import jax.numpy as jnp
from jax import lax
from jax.experimental import pallas as pl
from jax.experimental.pallas import tpu as pltpu
from jax.experimental.pallas import tpu_sc as plsc

N = 10000
DF = 256
HID = 256
HEADS = 3
OUT = 256
E_RAW = 160000
E_TOT = E_RAW + N

NC, NS = 2, 16
NW = NC * NS

PER = 5376
EP = PER * NW
BK = 64
NB = PER // BK
NI = PER // 16

PERF = 5120
EPF = PERF * NW
NBF = PERF // BK

ROWS_PT = 632
ROWS_TAIL = N - (NS - 1) * ROWS_PT


def _pad128(x):
    return (x + 127) // 128 * 128


def _mesh():
    return plsc.VectorSubcoreMesh(core_axis_name="c", subcore_axis_name="s")


def _mm_logits(x, W, As, Ad, heads, ch, rb=1000):
    m, k = x.shape
    M = W.shape[1]

    def body(x_ref, w_ref, as_ref, ad_ref, *outs):
        h = jnp.dot(x_ref[...], w_ref[...], preferred_element_type=jnp.float32)
        for j in range(ch):
            outs[j][...] = h[:, j * 128:(j + 1) * 128]
        outs[ch][...] = jnp.dot(h, as_ref[...], preferred_element_type=jnp.float32)
        outs[ch + 1][...] = jnp.dot(h, ad_ref[...], preferred_element_type=jnp.float32)

    res = pl.pallas_call(
        body,
        grid=(m // rb,),
        in_specs=[pl.BlockSpec((rb, k), lambda i: (i, 0)),
                  pl.BlockSpec((k, M), lambda i: (0, 0)),
                  pl.BlockSpec((M, heads), lambda i: (0, 0)),
                  pl.BlockSpec((M, heads), lambda i: (0, 0))],
        out_specs=[pl.BlockSpec((rb, 128), lambda i: (i, 0))] * ch
        + [pl.BlockSpec((rb, heads), lambda i: (i, 0))] * 2,
        out_shape=[jax.ShapeDtypeStruct((m, 128), jnp.float32)] * ch
        + [jax.ShapeDtypeStruct((m, heads), jnp.float32)] * 2,
    )(x, W, As, Ad)
    return list(res[:ch]), res[ch], res[ch + 1]


def _mm2_logits(parts, b1c, W2, As, Ad, rb=1000):
    def body(*refs):
        prefs = refs[:6]
        b_ref, w_ref, as_ref, ad_ref = refs[6:10]
        outs = refs[10:]
        acc = jnp.zeros((rb, OUT), jnp.float32)
        for p in range(6):
            xp = prefs[p][0] + prefs[p][1] + b_ref[p][None, :]
            xp = jnp.maximum(xp, 0.0)
            acc = acc + jnp.dot(xp, w_ref[p * 128:(p + 1) * 128, :],
                                preferred_element_type=jnp.float32)
        for j in range(2):
            outs[j][...] = acc[:, j * 128:(j + 1) * 128]
        outs[2][...] = jnp.dot(acc, as_ref[...], preferred_element_type=jnp.float32)
        outs[3][...] = jnp.dot(acc, ad_ref[...], preferred_element_type=jnp.float32)

    res = pl.pallas_call(
        body,
        grid=(N // rb,),
        in_specs=[pl.BlockSpec((NC, rb, 128), lambda i: (0, i, 0))] * 6
        + [pl.BlockSpec((6, 128), lambda i: (0, 0)),
           pl.BlockSpec((HEADS * HID, OUT), lambda i: (0, 0)),
           pl.BlockSpec((OUT, 1), lambda i: (0, 0)),
           pl.BlockSpec((OUT, 1), lambda i: (0, 0))],
        out_specs=[pl.BlockSpec((rb, 128), lambda i: (i, 0))] * 2
        + [pl.BlockSpec((rb, 1), lambda i: (i, 0))] * 2,
        out_shape=[jax.ShapeDtypeStruct((N, 128), jnp.float32)] * 2
        + [jax.ShapeDtypeStruct((N, 1), jnp.float32)] * 2,
    )(*parts, b1c, W2, As, Ad)
    return list(res[:2]), res[2], res[3]


def _den_recip(parts):
    nh = parts.shape[1]

    def body(p_ref, o_ref):
        s = jnp.sum(p_ref[...], axis=0)
        o_ref[...] = 1.0 / (s + 1e-16)

    return pl.pallas_call(
        body,
        in_specs=[pl.BlockSpec((NW, nh), lambda: (0, 0))],
        out_specs=pl.BlockSpec((nh,), lambda: (0,)),
        out_shape=jax.ShapeDtypeStruct((nh,), jnp.float32),
    )(parts)


def _assemble_h2(parts, b2c, rb=1000):
    def body(p0, p1, b_ref, o_ref):
        o_ref[:, 0:128] = p0[0] + p0[1] + b_ref[0][None, :]
        o_ref[:, 128:256] = p1[0] + p1[1] + b_ref[1][None, :]

    return pl.pallas_call(
        body,
        grid=(N // rb,),
        in_specs=[pl.BlockSpec((NC, rb, 128), lambda i: (0, i, 0))] * 2
        + [pl.BlockSpec((2, 128), lambda i: (0, 0))],
        out_specs=pl.BlockSpec((rb, OUT), lambda i: (i, 0)),
        out_shape=jax.ShapeDtypeStruct((N, OUT), jnp.float32),
    )(parts[0], parts[1], b2c)


def _sc_edge_softmax_num(es_f, ed_f, srcp, dstp, zeros_nh, heads):
    nh = N * heads
    nhp = _pad128(nh)

    @functools.partial(
        pl.kernel, mesh=_mesh(),
        compiler_params=pltpu.CompilerParams(needs_layout_passes=False),
        out_type=[jax.ShapeDtypeStruct((heads * EP,), jnp.float32),
                  jax.ShapeDtypeStruct((NW * nhp,), jnp.float32)],
        scratch_types=[pltpu.VMEM((nh,), jnp.float32),
                       pltpu.VMEM((nh,), jnp.float32),
                       pltpu.VMEM((nhp,), jnp.float32),
                       pltpu.VMEM((PER,), jnp.int32),
                       pltpu.VMEM((PER,), jnp.int32),
                       pltpu.VMEM((heads * PER,), jnp.float32)],
    )
    def k(es_hbm, ed_hbm, src_hbm, dst_hbm, z_hbm, ex_hbm, dp_hbm,
          es_t, ed_t, den_t, src_v, dst_v, ex_v):
        wid = lax.axis_index("s") * NC + lax.axis_index("c")
        base = wid * PER
        pltpu.sync_copy(es_hbm, es_t)
        pltpu.sync_copy(ed_hbm, ed_t)
        pltpu.sync_copy(z_hbm, den_t)
        pltpu.sync_copy(src_hbm.at[pl.ds(base, PER)], src_v)
        pltpu.sync_copy(dst_hbm.at[pl.ds(base, PER)], dst_v)

        def body(i, carry):
            s = src_v[pl.ds(i * 16, 16)]
            d = dst_v[pl.ds(i * 16, 16)]
            gid = base + i * 16 + lax.iota(jnp.int32, 16)
            valid = gid < E_TOT
            for h in range(heads):
                es_g = plsc.load_gather(es_t, [s * heads + h])
                ed_g = plsc.load_gather(ed_t, [d * heads + h])
                e = es_g + ed_g
                e = jnp.maximum(e, 0.2 * e)
                ex = jnp.where(valid, jnp.exp(e), 0.0)
                ex_v[pl.ds(h * PER + i * 16, 16)] = ex
                plsc.addupdate_scatter(den_t, [d * heads + h], ex)
            return carry

        lax.fori_loop(0, NI, body, 0)
        for h in range(heads):
            pltpu.sync_copy(ex_v.at[pl.ds(h * PER, PER)],
                            ex_hbm.at[pl.ds(h * EP + base, PER)])
        pltpu.sync_copy(den_t, dp_hbm.at[pl.ds(wid * nhp, nhp)])

    return k(es_f, ed_f, srcp, dstp, zeros_nh)


def _sc_alpha(ex_f, recip_f, dstp, heads):
    nhp = recip_f.shape[0]

    @functools.partial(
        pl.kernel, mesh=_mesh(),
        compiler_params=pltpu.CompilerParams(needs_layout_passes=False),
        out_type=jax.ShapeDtypeStruct((heads * EP,), jnp.float32),
        scratch_types=[pltpu.VMEM((nhp,), jnp.float32),
                       pltpu.VMEM((PER,), jnp.int32),
                       pltpu.VMEM((heads * PER,), jnp.float32)],
    )
    def k(ex_hbm, recip_hbm, dst_hbm, al_hbm, recip_t, dst_v, ex_v):
        wid = lax.axis_index("s") * NC + lax.axis_index("c")
        base = wid * PER
        pltpu.sync_copy(recip_hbm, recip_t)
        pltpu.sync_copy(dst_hbm.at[pl.ds(base, PER)], dst_v)
        for h in range(heads):
            pltpu.sync_copy(ex_hbm.at[pl.ds(h * EP + base, PER)],
                            ex_v.at[pl.ds(h * PER, PER)])

        def abody(i, carry):
            d = dst_v[pl.ds(i * 16, 16)]
            for h in range(heads):
                rd = plsc.load_gather(recip_t, [d * heads + h])
                ex_v[pl.ds(h * PER + i * 16, 16)] = \
                    ex_v[pl.ds(h * PER + i * 16, 16)] * rd
            return carry

        lax.fori_loop(0, NI, abody, 0)
        for h in range(heads):
            pltpu.sync_copy(ex_v.at[pl.ds(h * PER, PER)],
                            al_hbm.at[pl.ds(h * EP + base, PER)])

    return k(ex_f, recip_f, dstp)


def _sc_aggregate(h_chunks, src2, dst2, alpha_f, zslab, heads, ch):
    @functools.partial(
        pl.kernel, mesh=_mesh(),
        compiler_params=pltpu.CompilerParams(needs_layout_passes=False),
        out_type=[jax.ShapeDtypeStruct((NC, N, 128), jnp.float32)
                  for _ in range(ch)],
        scratch_types=[pltpu.VMEM_SHARED((N, 128), jnp.float32),
                       pltpu.VMEM((NB, BK), jnp.int32),
                       pltpu.VMEM((NB, BK), jnp.int32),
                       pltpu.VMEM((PER // 2,), jnp.float32),
                       pltpu.VMEM((BK, 128), jnp.float32),
                       pltpu.VMEM((BK, 128), jnp.float32),
                       pltpu.VMEM((BK, 128), jnp.float32),
                       pltpu.SemaphoreType.DMA,
                       pltpu.SemaphoreType.DMA,
                       pltpu.SemaphoreType.DMA,
                       pltpu.SemaphoreType.DMA,
                       pltpu.SemaphoreType.DMA,
                       pltpu.SemaphoreType.DMA],
    )
    def k(*refs):
        hs = refs[:ch]
        src2_h, dst2_h, al_h, z_h = refs[ch:ch + 4]
        outs = refs[ch + 4:2 * ch + 4]
        (slab, src2_v, dst2_v, alpha_v, rows0, rows1, rows2,
         g0, g1, g2, s0, s1, s2) = refs[2 * ch + 4:]
        bufs = (rows0, rows1, rows2)
        gsems = (g0, g1, g2)
        ssems = (s0, s1, s2)
        cid = lax.axis_index("c")
        sid = lax.axis_index("s")
        wid = sid * NC + cid
        base = wid * PER
        pltpu.sync_copy(src2_h.at[wid], src2_v)
        pltpu.sync_copy(dst2_h.at[wid], dst2_v)

        r0 = sid * ROWS_PT
        HALF = PER // 2
        for c in range(ch):
            h0 = c // 2
            pltpu.sync_copy(al_h.at[pl.ds(h0 * EP + base, HALF)], alpha_v)

            def gather(b, kbuf):
                pltpu.async_copy(hs[c].at[src2_v.at[b]], bufs[kbuf],
                                 gsems[kbuf])

            def wait_gather(kbuf):
                pltpu.make_async_copy(z_h.at[pl.ds(0, BK), :], bufs[kbuf],
                                      gsems[kbuf]).wait()

            def scatter(b, kbuf):
                pltpu.async_copy(bufs[kbuf], slab.at[dst2_v.at[b]],
                                 ssems[kbuf], add=True)

            def wait_scatter(kbuf):
                pltpu.make_async_copy(bufs[kbuf], slab.at[pl.ds(0, BK), :],
                                      ssems[kbuf]).wait()

            def scale(b, kbuf):
                buf = bufs[kbuf]

                rel = lax.rem(b * BK, HALF)

                @plsc.parallel_loop(0, BK // 16, 1, unroll=2)
                def _(r16):
                    a_vec = alpha_v[pl.ds(rel + r16 * 16, 16)]
                    for rr in range(16):
                        row = r16 * 16 + rr
                        av = jnp.full((16,), a_vec[rr], jnp.float32)
                        for j in range(8):
                            buf[row, pl.ds(j * 16, 16)] = \
                                buf[row, pl.ds(j * 16, 16)] * av

            @pl.when(sid < NS - 1)
            def _():
                pltpu.sync_copy(z_h.at[pl.ds(r0, ROWS_PT), :],
                                slab.at[pl.ds(r0, ROWS_PT), :])

            @pl.when(sid == NS - 1)
            def _():
                pltpu.sync_copy(z_h.at[pl.ds(r0, ROWS_TAIL), :],
                                slab.at[pl.ds(r0, ROWS_TAIL), :])

            plsc.subcore_barrier()

            gather(0, 0)

            def pbody(t, carry):
                @pl.when(t == NB // 3 // 2)
                def _():
                    pltpu.sync_copy(
                        al_h.at[pl.ds(h0 * EP + base + HALF, HALF)], alpha_v)

                for ph in range(3):
                    b = 3 * t + ph
                    k1 = (ph + 1) % 3
                    wait_gather(ph)
                    if ph == 2:
                        wait_scatter(k1)

                        @pl.when(t < NB // 3 - 1)
                        def _():
                            gather(b + 1, k1)
                    else:

                        @pl.when(t > 0)
                        def _():
                            wait_scatter(k1)

                        gather(b + 1, k1)
                    scale(b, ph)
                    scatter(b, ph)
                return carry

            lax.fori_loop(0, NB // 3, pbody, 0)
            wait_scatter(1)
            wait_scatter(2)
            plsc.subcore_barrier()

            @pl.when(sid < NS - 1)
            def _():
                pltpu.sync_copy(slab.at[pl.ds(r0, ROWS_PT), :],
                                outs[c].at[cid, pl.ds(r0, ROWS_PT), :])

            @pl.when(sid == NS - 1)
            def _():
                pltpu.sync_copy(slab.at[pl.ds(r0, ROWS_TAIL), :],
                                outs[c].at[cid, pl.ds(r0, ROWS_TAIL), :])

            plsc.subcore_barrier()

    return k(*h_chunks, src2, dst2, alpha_f, zslab)


def _sc_final(h2, e0r, e1r):
    @functools.partial(
        pl.kernel, mesh=_mesh(),
        compiler_params=pltpu.CompilerParams(needs_layout_passes=False),
        out_type=jax.ShapeDtypeStruct((E_RAW,), jnp.float32),
        scratch_types=[pltpu.VMEM((NBF, BK), jnp.int32),
                       pltpu.VMEM((NBF, BK), jnp.int32),
                       pltpu.VMEM((BK, OUT), jnp.float32),
                       pltpu.VMEM((BK, OUT), jnp.float32),
                       pltpu.VMEM((BK, OUT), jnp.float32),
                       pltpu.VMEM((BK, OUT), jnp.float32),
                       pltpu.VMEM((PERF,), jnp.float32),
                       pltpu.VMEM((256,), jnp.float32),
                       pltpu.SemaphoreType.DMA,
                       pltpu.SemaphoreType.DMA],
    )
    def k(h2_h, e0_h, e1_h, o_h, e0_v, e1_v, ra0, rb0, ra1, rb1,
          res_v, tbuf, sem0, sem1):
        wid = lax.axis_index("s") * NC + lax.axis_index("c")
        abufs = (ra0, ra1)
        bbufs = (rb0, rb1)
        sems = (sem0, sem1)
        pltpu.sync_copy(e0_h.at[wid], e0_v)
        pltpu.sync_copy(e1_h.at[wid], e1_v)
        lane = lax.iota(jnp.int32, 16)

        def gather(b, kbuf):
            pltpu.async_copy(h2_h.at[e0_v.at[b]], abufs[kbuf], sems[kbuf])
            pltpu.async_copy(h2_h.at[e1_v.at[b]], bbufs[kbuf], sems[kbuf])

        def wait_pair(kbuf):
            pltpu.make_async_copy(h2_h.at[pl.ds(0, BK)], abufs[kbuf],
                                  sems[kbuf]).wait()
            pltpu.make_async_copy(h2_h.at[pl.ds(0, BK)], bbufs[kbuf],
                                  sems[kbuf]).wait()

        def dot(b, kbuf):
            av = abufs[kbuf]
            bv = bbufs[kbuf]

            def rbody(r16, c2):
                for rr in range(16):
                    row = r16 * 16 + rr
                    acc = jnp.zeros((16,), jnp.float32)
                    for j in range(OUT // 16):
                        acc = acc + (av[row, pl.ds(j * 16, 16)] *
                                     bv[row, pl.ds(j * 16, 16)])
                    sidx = lane * 16 + lax.rem(lane + rr, 16)
                    plsc.store_scatter(tbuf, [sidx], acc)
                resvec = jnp.zeros((16,), jnp.float32)
                for l in range(16):
                    gidx = l * 16 + lax.rem(lane + l, 16)
                    resvec = resvec + plsc.load_gather(tbuf, [gidx])
                res_v[pl.ds(b * BK + r16 * 16, 16)] = resvec
                return c2

            lax.fori_loop(0, BK // 16, rbody, 0)

        gather(0, 0)

        def pbody(i, carry):
            b0 = 2 * i
            b1 = 2 * i + 1
            wait_pair(0)
            gather(b1, 1)
            dot(b0, 0)
            wait_pair(1)

            @pl.when(i < NBF // 2 - 1)
            def _():
                gather(b0 + 2, 0)

            dot(b1, 1)
            return carry

        lax.fori_loop(0, NBF // 2, pbody, 0)

        @plsc.parallel_loop(0, PERF // 16, 1, unroll=4)
        def _(i):
            x = res_v[pl.ds(i * 16, 16)]
            res_v[pl.ds(i * 16, 16)] = 1.0 / (1.0 + jnp.exp(-x))

        @pl.when(wid < NW - 1)
        def _():
            pltpu.sync_copy(res_v, o_h.at[pl.ds(wid * PERF, PERF)])

        @pl.when(wid == NW - 1)
        def _():
            tail = E_RAW - (NW - 1) * PERF
            pltpu.sync_copy(res_v.at[pl.ds(0, tail)],
                            o_h.at[pl.ds((NW - 1) * PERF, tail)])

    return k(h2, e0r, e1r)


def kernel(Features, A, E, W1, a1_src, a1_dst, b1, W2, a2_src, a2_dst, b2):
    i32 = jnp.int32
    loops = jnp.arange(N, dtype=i32)
    zpad = jnp.zeros((EP - E_TOT,), i32)
    srcp = jnp.concatenate([A[0].astype(i32), loops, zpad])
    dstp = jnp.concatenate([A[1].astype(i32), loops, zpad])
    src2 = srcp.reshape(NW, NB, BK)
    dst2 = dstp.reshape(NW, NB, BK)

    eye = jnp.eye(HEADS, dtype=jnp.float32)
    As1 = (a1_src[:, :, None] * eye[:, None, :]).reshape(HEADS * HID, HEADS)
    Ad1 = (a1_dst[:, :, None] * eye[:, None, :]).reshape(HEADS * HID, HEADS)
    As2 = a2_src.reshape(OUT, 1)
    Ad2 = a2_dst.reshape(OUT, 1)

    zeros_nh1 = jnp.zeros((_pad128(N * HEADS),), jnp.float32)
    zeros_nh2 = jnp.zeros((_pad128(N),), jnp.float32)
    zslab = jnp.zeros((N, 128), jnp.float32)

    hc1, es1, ed1 = _mm_logits(Features, W1, As1, Ad1, HEADS, 6)
    ex1, dp1 = _sc_edge_softmax_num(es1.reshape(-1), ed1.reshape(-1),
                                    srcp, dstp, zeros_nh1, HEADS)
    recip1 = _den_recip(dp1.reshape(NW, _pad128(N * HEADS)))
    alpha1 = _sc_alpha(ex1, recip1, dstp, HEADS)
    parts1 = _sc_aggregate(hc1, src2, dst2, alpha1, zslab, HEADS, 6)

    hc2, es2, ed2 = _mm2_logits(parts1, b1.reshape(6, 128), W2, As2, Ad2)
    ex2, dp2 = _sc_edge_softmax_num(es2.reshape(-1), ed2.reshape(-1),
                                    srcp, dstp, zeros_nh2, 1)
    recip2 = _den_recip(dp2.reshape(NW, _pad128(N)))
    alpha2 = _sc_alpha(ex2, recip2, dstp, 1)
    parts2 = _sc_aggregate(hc2, src2, dst2, alpha2, zslab, 1, 2)
    h2 = _assemble_h2(parts2, b2.reshape(2, 128))

    fpad = jnp.zeros((EPF - E_RAW,), i32)
    e0r = jnp.concatenate([E[0].astype(i32), fpad]).reshape(NW, NBF, BK)
    e1r = jnp.concatenate([E[1].astype(i32), fpad]).reshape(NW, NBF, BK)
    return _sc_final(h2, e0r, e1r)
